# baseline (speedup 1.0000x reference)
import jax
import jax.numpy as jnp
from jax.experimental import pallas as pl
from jax.experimental.pallas import tpu as pltpu

_BIT = 512
_SPIKE = 7.0

_RED_ROWS = 2352
_RED_COLS = 8192
_RED_BLOCK = 112
_OUT_ROWS = 384
_OUT_BLOCK = 24


def _minmax_body(x_ref, mm_ref):
    j = pl.program_id(0)
    bmin = jnp.min(x_ref[...])
    bmax = jnp.max(x_ref[...])

    @pl.when(j == 0)
    def _init():
        mm_ref[0] = bmin
        mm_ref[1] = bmax

    @pl.when(j > 0)
    def _acc():
        mm_ref[0] = jnp.minimum(mm_ref[0], bmin)
        mm_ref[1] = jnp.maximum(mm_ref[1], bmax)


def _bcast_body(y_ref, mm_ref, o_ref, vals_ref):
    j = pl.program_id(0)

    @pl.when(j == 0)
    def _quantize():
        y = y_ref[...]
        ymax = jnp.max(y)
        ymin = jnp.min(y)
        v = y / (ymax - ymin) * _SPIKE * float(_BIT)
        kf = jnp.clip(v / _SPIKE + 0.5, 0.0, float(_BIT - 1))
        k0 = kf.astype(jnp.int32)
        km = jnp.maximum(k0 - 1, 0)
        kp = jnp.minimum(k0 + 1, _BIT - 1)

        def dist(k):
            return jnp.abs(v - k.astype(jnp.float32) * _SPIKE)

        dm = dist(km)
        d0 = dist(k0)
        dp = dist(kp)
        best = km
        bd = dm
        t0 = d0 < bd
        best = jnp.where(t0, k0, best)
        bd = jnp.where(t0, d0, bd)
        tp = dp < bd
        best = jnp.where(tp, kp, best)
        vals = best.astype(jnp.float32) * _SPIKE
        n = (mm_ref[1] - mm_ref[0]).astype(jnp.int32) + 1
        row = jax.lax.broadcasted_iota(jnp.int32, v.shape, 0)
        vals_ref[...] = jnp.where(row < n, vals, 0.0)

    o_ref[...] = jnp.broadcast_to(vals_ref[...][None], o_ref.shape)


def kernel(x, y):
    x2 = x.reshape(_RED_ROWS, _RED_COLS)
    mm = pl.pallas_call(
        _minmax_body,
        grid=(_RED_ROWS // _RED_BLOCK,),
        in_specs=[pl.BlockSpec((_RED_BLOCK, _RED_COLS), lambda j: (j, 0))],
        out_specs=pl.BlockSpec(memory_space=pltpu.SMEM),
        out_shape=jax.ShapeDtypeStruct((2,), jnp.float32),
    )(x2)

    return jnp.broadcast_to(mm[0], x.shape) + jnp.zeros(x.shape, jnp.float32)


# speedup vs baseline: 2.7268x; 2.7268x over previous
import jax
import jax.numpy as jnp
from jax.experimental import pallas as pl
from jax.experimental.pallas import tpu as pltpu

_BIT = 512
_SPIKE = 7.0

_RED_ROWS = 2352
_RED_COLS = 8192
_RED_BLOCK = 112
_OUT_ROWS = 384
_OUT_BLOCK = 24


def _minmax_body(x_ref, mm_ref):
    j = pl.program_id(0)
    bmin = jnp.min(x_ref[...])
    bmax = jnp.max(x_ref[...])

    @pl.when(j == 0)
    def _init():
        mm_ref[0] = bmin
        mm_ref[1] = bmax

    @pl.when(j > 0)
    def _acc():
        mm_ref[0] = jnp.minimum(mm_ref[0], bmin)
        mm_ref[1] = jnp.maximum(mm_ref[1], bmax)


def _bcast_body(y_ref, mm_ref, o_ref, vals_ref):
    j = pl.program_id(0)

    @pl.when(j == 0)
    def _quantize():
        y = y_ref[...]
        ymax = jnp.max(y)
        ymin = jnp.min(y)
        v = y / (ymax - ymin) * _SPIKE * float(_BIT)
        kf = jnp.clip(v / _SPIKE + 0.5, 0.0, float(_BIT - 1))
        k0 = kf.astype(jnp.int32)
        km = jnp.maximum(k0 - 1, 0)
        kp = jnp.minimum(k0 + 1, _BIT - 1)

        def dist(k):
            return jnp.abs(v - k.astype(jnp.float32) * _SPIKE)

        dm = dist(km)
        d0 = dist(k0)
        dp = dist(kp)
        best = km
        bd = dm
        t0 = d0 < bd
        best = jnp.where(t0, k0, best)
        bd = jnp.where(t0, d0, bd)
        tp = dp < bd
        best = jnp.where(tp, kp, best)
        vals = best.astype(jnp.float32) * _SPIKE
        n = (mm_ref[1] - mm_ref[0]).astype(jnp.int32) + 1
        row = jax.lax.broadcasted_iota(jnp.int32, v.shape, 0)
        vals_ref[...] = jnp.where(row < n, vals, 0.0)

    o_ref[...] = jnp.broadcast_to(vals_ref[...][None], o_ref.shape)


def kernel(x, y):
    mm = jnp.stack([jnp.min(x), jnp.max(x)])

    out3 = pl.pallas_call(
        _bcast_body,
        grid=(_OUT_ROWS // _OUT_BLOCK,),
        in_specs=[
            pl.BlockSpec((224, 1), lambda j: (0, 0)),
            pl.BlockSpec(memory_space=pltpu.SMEM),
        ],
        out_specs=pl.BlockSpec((_OUT_BLOCK, 224, 224), lambda j: (j, 0, 0)),
        out_shape=jax.ShapeDtypeStruct((_OUT_ROWS, 224, 224), jnp.float32),
        scratch_shapes=[pltpu.VMEM((224, 1), jnp.float32)],
    )(y.reshape(224, 1), mm)
    return out3.reshape(x.shape)
